# baseline (speedup 1.0000x reference)
import functools

import jax
import jax.numpy as jnp
from jax.experimental import pallas as pl
from jax.experimental.pallas import tpu as pltpu

B = 16
S = 4096
D = 768
MAX_ITEMS = 1366
NUM_LAYERS = 3
EPS = 1e-5
BLK = 2048
NBLK = S // BLK
CROWS = BLK + S + 16


def _prep_kernel(ipw_ref, tmp_ref, lw_ref, mask_ref, cpad_ref, start_ref):
    ipt = (ipw_ref[...] + tmp_ref[...]).astype(jnp.bfloat16)
    lwb = lw_ref[...].astype(jnp.bfloat16)
    r = jax.lax.broadcasted_iota(jnp.int32, (S, MAX_ITEMS), 0)
    k = jax.lax.broadcasted_iota(jnp.int32, (S, MAX_ITEMS), 1)
    e_item = (r // 3 == k).astype(jnp.bfloat16)
    r3 = jax.lax.broadcasted_iota(jnp.int32, (S, NUM_LAYERS), 0)
    j3 = jax.lax.broadcasted_iota(jnp.int32, (S, NUM_LAYERS), 1)
    e_layer = (r3 % 3 == j3).astype(jnp.bfloat16)
    dn = (((1,), (0,)), ((), ()))
    cflat = (jax.lax.dot_general(e_item, ipt, dn,
                                 preferred_element_type=jnp.float32)
             + jax.lax.dot_general(e_layer, lwb, dn,
                                   preferred_element_type=jnp.float32))
    cpad_ref[0:BLK, :] = jnp.zeros((BLK, D), jnp.bfloat16)
    cpad_ref[BLK:BLK + S, :] = cflat.astype(jnp.bfloat16)
    cpad_ref[BLK + S:CROWS, :] = jnp.zeros((16, D), jnp.bfloat16)
    pos = jax.lax.broadcasted_iota(jnp.int32, (B, S), 1)
    masked_pos = jnp.where(mask_ref[...] > 0, pos, jnp.int32(S))
    start_ref[...] = jnp.min(masked_pos, axis=1, keepdims=True)


def _window_off(start_ref, lin):
    b = lin // NBLK
    j = lin - b * NBLK
    s0 = start_ref[b, 0]
    return jnp.clip(j * BLK - s0, -BLK, S - BLK) + BLK


def _main_kernel(tok_ref, maskt_ref, cpad_ref, start_ref, lnw_ref, lnb_ref,
                 out_ref):
    b = pl.program_id(0)
    j = pl.program_id(1)
    off = _window_off(start_ref, b * NBLK + j)
    off16 = pl.multiple_of((off // 16) * 16, 16)
    r = off - off16
    big = cpad_ref[pl.ds(off16, BLK + 16), :].astype(jnp.float32)
    shift16 = jax.lax.rem(16 - r, 16)
    rot = pltpu.roll(big.reshape((BLK + 16) // 16, 16, D), shift16, 1)
    rot = rot.reshape(BLK + 16, D)
    t = jax.lax.rem(jax.lax.broadcasted_iota(jnp.int32, (BLK, 1), 0), 16)
    cw = jnp.where(t < 16 - r, rot[0:BLK], rot[16:BLK + 16])
    enh = tok_ref[0] + cw
    s1 = jnp.sum(enh, axis=1, keepdims=True)
    s2 = jnp.sum(enh * enh, axis=1, keepdims=True)
    mean = s1 * (1.0 / D)
    var = s2 * (1.0 / D) - mean * mean
    lane = jax.lax.broadcasted_iota(jnp.int32, (BLK, B), 1)
    m = jnp.sum(jnp.where(lane == b, maskt_ref[...], 0.0), axis=1,
                keepdims=True)
    inv = jax.lax.rsqrt(var + EPS) * m
    out_ref[0] = (enh - mean) * inv * lnw_ref[...] + lnb_ref[...] * m


@functools.partial(jax.jit, static_argnames=("interpret",))
def kernel(token_embeddings, attention_mask, item_pos_w, layer_w, temporal,
           ln_w, ln_b, interpret=False):
    mask = attention_mask.astype(jnp.int32)

    cpad, start = pl.pallas_call(
        _prep_kernel,
        out_shape=(
            jax.ShapeDtypeStruct((CROWS, D), jnp.bfloat16),
            jax.ShapeDtypeStruct((B, 1), jnp.int32),
        ),
        interpret=interpret,
    )(item_pos_w, temporal, layer_w, mask)

    maskt = mask.astype(jnp.float32).T
    lnw = ln_w.reshape(1, D)
    lnb = ln_b.reshape(1, D)

    out = pl.pallas_call(
        _main_kernel,
        grid=(B, NBLK),
        in_specs=[
            pl.BlockSpec((1, BLK, D), lambda b, j: (b, j, 0)),
            pl.BlockSpec((BLK, B), lambda b, j: (j, 0)),
            pl.BlockSpec((CROWS, D), lambda b, j: (0, 0)),
            pl.BlockSpec(memory_space=pltpu.SMEM),
            pl.BlockSpec((1, D), lambda b, j: (0, 0)),
            pl.BlockSpec((1, D), lambda b, j: (0, 0)),
        ],
        out_specs=pl.BlockSpec((1, BLK, D), lambda b, j: (b, j, 0)),
        out_shape=jax.ShapeDtypeStruct((B, S, D), jnp.float32),
        interpret=interpret,
    )(token_embeddings, maskt, cpad, start, lnw, lnb)
    return out


# speedup vs baseline: 1.0569x; 1.0569x over previous
import functools

import jax
import jax.numpy as jnp
from jax.experimental import pallas as pl
from jax.experimental.pallas import tpu as pltpu

B = 16
S = 4096
D = 768
MAX_ITEMS = 1366
NUM_LAYERS = 3
EPS = 1e-5
BLK = 2048
NBLK = S // BLK
CROWS = BLK + S + 8


def _prep_kernel(ipw_ref, tmp_ref, lw_ref, mask_ref, cpad_ref, start_ref):
    ipt = (ipw_ref[...] + tmp_ref[...]).astype(jnp.bfloat16)
    lwb = lw_ref[...].astype(jnp.bfloat16)
    dn = (((1,), (0,)), ((), ()))
    cpad_ref[0:BLK, :] = jnp.zeros((BLK, D), jnp.float32)
    CH = 512
    for g in range(S // CH):
        k0 = (CH * g) // 3
        k1 = min((CH * (g + 1) - 1) // 3 + 1, MAX_ITEMS)
        kw = -(-(k1 - k0) // 8) * 8
        k0 = min(k0, MAX_ITEMS - kw)
        rr = CH * g + jax.lax.broadcasted_iota(jnp.int32, (CH, kw), 0)
        kk = k0 + jax.lax.broadcasted_iota(jnp.int32, (CH, kw), 1)
        e_item = (rr // 3 == kk).astype(jnp.bfloat16)
        r3 = jax.lax.broadcasted_iota(jnp.int32, (CH, NUM_LAYERS), 0)
        j3 = jax.lax.broadcasted_iota(jnp.int32, (CH, NUM_LAYERS), 1)
        e_layer = ((CH * g + r3) % 3 == j3).astype(jnp.bfloat16)
        chunk = (jax.lax.dot_general(e_item, ipt[k0:k0 + kw], dn,
                                     preferred_element_type=jnp.float32)
                 + jax.lax.dot_general(e_layer, lwb, dn,
                                       preferred_element_type=jnp.float32))
        cpad_ref[BLK + CH * g:BLK + CH * (g + 1), :] = chunk
    cpad_ref[BLK + S:CROWS, :] = jnp.zeros((8, D), jnp.float32)
    pos = jax.lax.broadcasted_iota(jnp.int32, (B, S), 1)
    masked_pos = jnp.where(mask_ref[...] > 0, pos, jnp.int32(S))
    start_ref[...] = jnp.min(masked_pos, axis=1, keepdims=True)


def _window_off(start_ref, lin):
    b = lin // NBLK
    j = lin - b * NBLK
    s0 = start_ref[b, 0]
    return jnp.clip(j * BLK - s0, -BLK, S - BLK) + BLK


def _main_kernel(tok_ref, maskt_ref, cpad_ref, start_ref, lnw_ref, lnb_ref,
                 out_ref):
    b = pl.program_id(0)
    j = pl.program_id(1)
    off = _window_off(start_ref, b * NBLK + j)
    off8 = pl.multiple_of((off // 8) * 8, 8)
    r = off - off8
    big = cpad_ref[pl.ds(off8, BLK + 8), :]
    shift8 = jax.lax.rem(8 - r, 8)
    rot = pltpu.roll(big.reshape((BLK + 8) // 8, 8, D), shift8, 1)
    rot = rot.reshape(BLK + 8, D)
    t = jax.lax.rem(jax.lax.broadcasted_iota(jnp.int32, (BLK, 1), 0), 8)
    cw = jnp.where(t < 8 - r, rot[0:BLK], rot[8:BLK + 8])
    enh = tok_ref[0] + cw
    s1 = jnp.sum(enh, axis=1, keepdims=True)
    s2 = jnp.sum(enh * enh, axis=1, keepdims=True)
    mean = s1 * (1.0 / D)
    var = s2 * (1.0 / D) - mean * mean
    lane = jax.lax.broadcasted_iota(jnp.int32, (BLK, B), 1)
    m = jnp.sum(jnp.where(lane == b, maskt_ref[...], 0.0), axis=1,
                keepdims=True)
    inv = jax.lax.rsqrt(var + EPS) * m
    out_ref[0] = (enh - mean) * inv * lnw_ref[...] + lnb_ref[...] * m


@functools.partial(jax.jit, static_argnames=("interpret",))
def kernel(token_embeddings, attention_mask, item_pos_w, layer_w, temporal,
           ln_w, ln_b, interpret=False):
    mask = attention_mask.astype(jnp.int32)

    cpad, start = pl.pallas_call(
        _prep_kernel,
        out_shape=(
            jax.ShapeDtypeStruct((CROWS, D), jnp.float32),
            jax.ShapeDtypeStruct((B, 1), jnp.int32),
        ),
        interpret=interpret,
    )(item_pos_w, temporal, layer_w, mask)

    maskt = mask.astype(jnp.float32).T
    lnw = ln_w.reshape(1, D)
    lnb = ln_b.reshape(1, D)

    out = pl.pallas_call(
        _main_kernel,
        grid=(B, NBLK),
        in_specs=[
            pl.BlockSpec((1, BLK, D), lambda b, j: (b, j, 0)),
            pl.BlockSpec((BLK, B), lambda b, j: (j, 0)),
            pl.BlockSpec((CROWS, D), lambda b, j: (0, 0)),
            pl.BlockSpec(memory_space=pltpu.SMEM),
            pl.BlockSpec((1, D), lambda b, j: (0, 0)),
            pl.BlockSpec((1, D), lambda b, j: (0, 0)),
        ],
        out_specs=pl.BlockSpec((1, BLK, D), lambda b, j: (b, j, 0)),
        out_shape=jax.ShapeDtypeStruct((B, S, D), jnp.float32),
        interpret=interpret,
    )(token_embeddings, maskt, cpad, start, lnw, lnb)
    return out


# speedup vs baseline: 1.0910x; 1.0322x over previous
import functools

import jax
import jax.numpy as jnp
from jax.experimental import pallas as pl
from jax.experimental.pallas import tpu as pltpu

B = 16
S = 4096
D = 768
MAX_ITEMS = 1366
NUM_LAYERS = 3
EPS = 1e-5
BLK = 2048
NBLK = S // BLK
CROWS = BLK + S + 8


def _start_kernel(mask_ref, start_ref):
    pos = jax.lax.broadcasted_iota(jnp.int32, (B, S), 1)
    masked_pos = jnp.where(mask_ref[...] > 0, pos, jnp.int32(S))
    start_ref[...] = jnp.min(masked_pos, axis=1, keepdims=True)


def _build_table(ipw_ref, tmp_ref, lw_ref, cpad):
    ipt = ipw_ref[...] + tmp_ref[...]
    lwb = lw_ref[...]
    dn = (((1,), (0,)), ((), ()))
    cpad[0:BLK, :] = jnp.zeros((BLK, D), jnp.float32)
    CH = 512
    for g in range(S // CH):
        k0 = (CH * g) // 3
        k1 = min((CH * (g + 1) - 1) // 3 + 1, MAX_ITEMS)
        kw = -(-(k1 - k0) // 8) * 8
        k0 = min(k0, MAX_ITEMS - kw)
        rr = CH * g + jax.lax.broadcasted_iota(jnp.int32, (CH, kw), 0)
        kk = k0 + jax.lax.broadcasted_iota(jnp.int32, (CH, kw), 1)
        e_item = (rr // 3 == kk).astype(jnp.bfloat16)
        r3 = jax.lax.broadcasted_iota(jnp.int32, (CH, NUM_LAYERS), 0)
        j3 = jax.lax.broadcasted_iota(jnp.int32, (CH, NUM_LAYERS), 1)
        e_layer = ((CH * g + r3) % 3 == j3).astype(jnp.bfloat16)
        chunk = (jax.lax.dot_general(e_item, ipt[k0:k0 + kw], dn,
                                     preferred_element_type=jnp.float32)
                 + jax.lax.dot_general(e_layer, lwb, dn,
                                       preferred_element_type=jnp.float32))
        cpad[BLK + CH * g:BLK + CH * (g + 1), :] = chunk
    cpad[BLK + S:CROWS, :] = jnp.zeros((8, D), jnp.float32)


def _window_off(start_ref, lin):
    b = lin // NBLK
    j = lin - b * NBLK
    s0 = start_ref[b, 0]
    return jnp.clip(j * BLK - s0, -BLK, S - BLK) + BLK


def _main_kernel(tok_ref, maskt_ref, ipw_ref, tmp_ref, lw_ref, start_ref,
                 lnw_ref, lnb_ref, out_ref, cpad):
    b = pl.program_id(0)
    j = pl.program_id(1)
    lin = b * NBLK + j

    @pl.when(lin == 0)
    def _build():
        _build_table(ipw_ref, tmp_ref, lw_ref, cpad)

    off = _window_off(start_ref, lin)
    off8 = pl.multiple_of((off // 8) * 8, 8)
    r = off - off8
    big = cpad[pl.ds(off8, BLK + 8), :]
    shift8 = jax.lax.rem(8 - r, 8)
    rot = pltpu.roll(big.reshape((BLK + 8) // 8, 8, D), shift8, 1)
    rot = rot.reshape(BLK + 8, D)
    t = jax.lax.rem(jax.lax.broadcasted_iota(jnp.int32, (BLK, 1), 0), 8)
    cw = jnp.where(t < 8 - r, rot[0:BLK], rot[8:BLK + 8])
    enh = tok_ref[0] + cw
    s1 = jnp.sum(enh, axis=1, keepdims=True)
    s2 = jnp.sum(enh * enh, axis=1, keepdims=True)
    mean = s1 * (1.0 / D)
    var = s2 * (1.0 / D) - mean * mean
    lane = jax.lax.broadcasted_iota(jnp.int32, (BLK, B), 1)
    m = jnp.sum(jnp.where(lane == b, maskt_ref[...], 0.0), axis=1,
                keepdims=True)
    inv = jax.lax.rsqrt(var + EPS) * m
    out_ref[0] = (enh - mean) * inv * lnw_ref[...] + lnb_ref[...] * m


@functools.partial(jax.jit, static_argnames=("interpret",))
def kernel(token_embeddings, attention_mask, item_pos_w, layer_w, temporal,
           ln_w, ln_b, interpret=False):
    mask = attention_mask.astype(jnp.int32)

    start = pl.pallas_call(
        _start_kernel,
        out_shape=jax.ShapeDtypeStruct((B, 1), jnp.int32),
        interpret=interpret,
    )(mask)

    maskt = mask.astype(jnp.float32).T
    lnw = ln_w.reshape(1, D)
    lnb = ln_b.reshape(1, D)
    ipw_bf = item_pos_w.astype(jnp.bfloat16)
    tmp_bf = temporal.astype(jnp.bfloat16)
    lw_bf = layer_w.astype(jnp.bfloat16)

    out = pl.pallas_call(
        _main_kernel,
        grid=(B, NBLK),
        in_specs=[
            pl.BlockSpec((1, BLK, D), lambda b, j: (b, j, 0)),
            pl.BlockSpec((BLK, B), lambda b, j: (j, 0)),
            pl.BlockSpec((MAX_ITEMS, D), lambda b, j: (0, 0)),
            pl.BlockSpec((MAX_ITEMS, D), lambda b, j: (0, 0)),
            pl.BlockSpec((NUM_LAYERS, D), lambda b, j: (0, 0)),
            pl.BlockSpec(memory_space=pltpu.SMEM),
            pl.BlockSpec((1, D), lambda b, j: (0, 0)),
            pl.BlockSpec((1, D), lambda b, j: (0, 0)),
        ],
        out_specs=pl.BlockSpec((1, BLK, D), lambda b, j: (b, j, 0)),
        out_shape=jax.ShapeDtypeStruct((B, S, D), jnp.float32),
        scratch_shapes=[pltpu.VMEM((CROWS, D), jnp.float32)],
        interpret=interpret,
    )(token_embeddings, maskt, ipw_bf, tmp_bf, lw_bf, start, lnw, lnb)
    return out
